# baseline (speedup 1.0000x reference)
import functools

import jax
import jax.numpy as jnp
from jax.experimental import pallas as pl
from jax.experimental.pallas import tpu as pltpu

N_NODES = 10000


def _gat(x, edge_index, W, att_src, att_dst, bias):
    N = x.shape[0]
    src = edge_index[0]
    dst = edge_index[1]
    loop = jnp.arange(N, dtype=src.dtype)
    src = jnp.concatenate([src, loop])
    dst = jnp.concatenate([dst, loop])
    h = x @ W
    a_src = (h * att_src).sum(-1)
    a_dst = (h * att_dst).sum(-1)
    alpha = jax.nn.leaky_relu(a_src[src] + a_dst[dst], negative_slope=0.2)
    amax = jax.ops.segment_max(alpha, dst, num_segments=N)
    alpha = jnp.exp(alpha - amax[dst])
    denom = jax.ops.segment_sum(alpha, dst, num_segments=N)
    alpha = alpha / (denom[dst] + 1e-16)
    msg = h[src] * alpha[:, None]
    out = jax.ops.segment_sum(msg, dst, num_segments=N)
    return out + bias


def _decoder_body(a_ref, b_ref, o_ref):
    prod = jax.lax.dot_general(
        a_ref[...], b_ref[...], (((1,), (1,)), ((), ())),
        preferred_element_type=jnp.float32)
    o_ref[...] = jax.nn.sigmoid(prod)


def _decoder(h):
    N, K = h.shape
    BM = BN = 512
    grid = (pl.cdiv(N, BM), pl.cdiv(N, BN))
    return pl.pallas_call(
        _decoder_body,
        grid=grid,
        in_specs=[
            pl.BlockSpec((BM, K), lambda i, j: (i, 0)),
            pl.BlockSpec((BN, K), lambda i, j: (j, 0)),
        ],
        out_specs=pl.BlockSpec((BM, BN), lambda i, j: (i, j)),
        out_shape=jax.ShapeDtypeStruct((N, N), jnp.float32),
    )(h, h)


def kernel(x, edge_index, W1, att_src1, att_dst1, b1, W2, att_src2, att_dst2, b2):
    h = jax.nn.relu(_gat(x, edge_index, W1, att_src1, att_dst1, b1))
    h = jax.nn.relu(_gat(h, edge_index, W2, att_src2, att_dst2, b2))
    adj = _decoder(h)
    return (adj, edge_index)


# speedup vs baseline: 1.0081x; 1.0081x over previous
import functools

import jax
import jax.numpy as jnp
from jax.experimental import pallas as pl
from jax.experimental.pallas import tpu as pltpu

N_NODES = 10000


def _gat(x, edge_index, W, att_src, att_dst, bias):
    N = x.shape[0]
    src = edge_index[0]
    dst = edge_index[1]
    loop = jnp.arange(N, dtype=src.dtype)
    src = jnp.concatenate([src, loop])
    dst = jnp.concatenate([dst, loop])
    h = x @ W
    a_src = (h * att_src).sum(-1)
    a_dst = (h * att_dst).sum(-1)
    alpha = jax.nn.leaky_relu(a_src[src] + a_dst[dst], negative_slope=0.2)
    amax = jax.ops.segment_max(alpha, dst, num_segments=N)
    alpha = jnp.exp(alpha - amax[dst])
    denom = jax.ops.segment_sum(alpha, dst, num_segments=N)
    alpha = alpha / (denom[dst] + 1e-16)
    msg = h[src] * alpha[:, None]
    out = jax.ops.segment_sum(msg, dst, num_segments=N)
    return out + bias


def _decoder_body(a_ref, b_ref, o_ref):
    prod = jax.lax.dot_general(
        a_ref[...], b_ref[...], (((1,), (1,)), ((), ())),
        preferred_element_type=jnp.float32)
    o_ref[...] = jax.nn.sigmoid(prod)


def _decoder(h):
    N, K = h.shape
    hb = h.astype(jnp.bfloat16)
    BM = BN = 512
    grid = (pl.cdiv(N, BM), pl.cdiv(N, BN))
    return pl.pallas_call(
        _decoder_body,
        grid=grid,
        in_specs=[
            pl.BlockSpec((BM, K), lambda i, j: (i, 0)),
            pl.BlockSpec((BN, K), lambda i, j: (j, 0)),
        ],
        out_specs=pl.BlockSpec((BM, BN), lambda i, j: (i, j)),
        out_shape=jax.ShapeDtypeStruct((N, N), jnp.float32),
    )(hb, hb)


def kernel(x, edge_index, W1, att_src1, att_dst1, b1, W2, att_src2, att_dst2, b2):
    h = jax.nn.relu(_gat(x, edge_index, W1, att_src1, att_dst1, b1))
    h = jax.nn.relu(_gat(h, edge_index, W2, att_src2, att_dst2, b2))
    adj = _decoder(h)
    return (adj, edge_index)


# speedup vs baseline: 2.4112x; 2.3919x over previous
import functools

import jax
import jax.numpy as jnp
from jax import lax
from jax.experimental import pallas as pl
from jax.experimental.pallas import tpu as pltpu
from jax.experimental.pallas import tpu_sc as plsc

N_NODES = 10000
E_REAL = 170000
NB = 42
E_PAD = 32 * NB * 128
_MESH = plsc.VectorSubcoreMesh(core_axis_name="c", subcore_axis_name="s")


@functools.partial(
    pl.kernel,
    mesh=_MESH,
    compiler_params=pltpu.CompilerParams(needs_layout_passes=False),
    out_type=(
        jax.ShapeDtypeStruct((32, NB, 128), jnp.float32),
        jax.ShapeDtypeStruct((N_NODES,), jnp.float32),
        jax.ShapeDtypeStruct((N_NODES,), jnp.float32),
    ),
    scratch_types=[
        pltpu.VMEM((N_NODES,), jnp.float32),
        pltpu.VMEM((N_NODES,), jnp.float32),
        pltpu.VMEM((16,), jnp.float32),
        pltpu.VMEM((NB, 128), jnp.int32),
        pltpu.VMEM((NB, 128), jnp.int32),
        pltpu.VMEM((NB, 128), jnp.float32),
        pltpu.VMEM((N_NODES,), jnp.float32),
        pltpu.VMEM_SHARED((N_NODES,), jnp.float32),
    ],
)
def _edge_softmax(a_src_h, a_dst_h, c_h, src_h, dst_h, p_out_h, den0_h, den1_h,
                  asrc_v, adst_v, c_v, src_v, dst_v, p_v, z_v, den_sh):
    core = lax.axis_index("c")
    sub = lax.axis_index("s")
    tid = core * 16 + sub

    pltpu.sync_copy(a_src_h, asrc_v)
    pltpu.sync_copy(a_dst_h, adst_v)
    pltpu.sync_copy(c_h, c_v)
    pltpu.sync_copy(src_h.at[tid], src_v)
    pltpu.sync_copy(dst_h.at[tid], dst_v)

    @pl.when(sub == 0)
    def _():
        def zb(i, _):
            z_v[pl.ds(i * 16, 16)] = jnp.zeros((16,), jnp.float32)
            return _
        lax.fori_loop(0, N_NODES // 16, zb, 0)
        pltpu.sync_copy(z_v, den_sh)

    cvec = c_v[...]
    base = tid * (NB * 128)

    def body(b, _):
        for j in range(8):
            sl = pl.ds(j * 16, 16)
            s_idx = src_v[b, sl]
            d_idx = dst_v[b, sl]
            av = plsc.load_gather(asrc_v, [s_idx])
            bv = plsc.load_gather(adst_v, [d_idx])
            al = av + bv
            al = jnp.where(al >= 0.0, al, al * jnp.float32(0.2))
            p = jnp.exp(al - cvec)
            eid = base + b * 128 + j * 16 + lax.iota(jnp.int32, 16)
            p = jnp.where(eid < E_REAL, p, jnp.float32(0.0))
            p_v[b, sl] = p
        return _

    lax.fori_loop(0, NB, body, 0)

    plsc.subcore_barrier()

    def addb(b, _):
        pltpu.sync_copy(p_v.at[b], den_sh.at[dst_v.at[b]], add=True)
        return _

    lax.fori_loop(0, NB, addb, 0)

    plsc.subcore_barrier()

    pltpu.sync_copy(p_v, p_out_h.at[tid])

    @pl.when(jnp.logical_and(sub == 0, core == 0))
    def _():
        pltpu.sync_copy(den_sh, den0_h)

    @pl.when(jnp.logical_and(sub == 0, core == 1))
    def _():
        pltpu.sync_copy(den_sh, den1_h)


NHALF = 5120
NROWS_ACC = 5248


def _make_agg(F):
    n_groups = F // 16

    @functools.partial(
        pl.kernel,
        mesh=_MESH,
        compiler_params=pltpu.CompilerParams(needs_layout_passes=False),
        out_type=jax.ShapeDtypeStruct((2 * NHALF, F), jnp.float32),
        scratch_types=[
            pltpu.VMEM((NB, 128), jnp.int32),
            pltpu.VMEM((NB, 128), jnp.int32),
            pltpu.VMEM((NB, 128), jnp.float32),
            pltpu.VMEM((128, F), jnp.float32),
            pltpu.VMEM((128,), jnp.int32),
            pltpu.VMEM((NROWS_ACC // 16, F), jnp.float32),
            pltpu.VMEM_SHARED((NROWS_ACC, F), jnp.float32),
            pltpu.SemaphoreType.DMA,
        ],
    )
    def agg(h_h, src_h, dst_h, p_h, out_h, src_v, dst_v, p_v, rows_v, idx_v,
            z_v, acc_sh, sem):
        core = lax.axis_index("c")
        sub = lax.axis_index("s")
        lo = core * NHALF
        zr = NROWS_ACC // 16

        def zrow(r, _):
            for j in range(n_groups):
                z_v[r, pl.ds(j * 16, 16)] = jnp.zeros((16,), jnp.float32)
            return _
        lax.fori_loop(0, zr, zrow, 0)
        pltpu.sync_copy(z_v, acc_sh.at[pl.ds(sub * zr, zr)])

        plsc.subcore_barrier()

        for ec in range(2):
            cid = sub * 2 + ec
            pltpu.sync_copy(src_h.at[cid], src_v)
            pltpu.sync_copy(dst_h.at[cid], dst_v)
            pltpu.sync_copy(p_h.at[cid], p_v)

            def batch(b, _):
                pltpu.async_copy(h_h.at[src_v.at[b]], rows_v, sem).wait()

                for g in range(8):
                    sl = pl.ds(g * 16, 16)
                    d = dst_v[b, sl]
                    loc = d - lo
                    inb = jnp.logical_and(loc >= 0, loc < NHALF)
                    idx_v[sl] = jnp.where(inb, loc, NHALF + g * 16
                                          + lax.iota(jnp.int32, 16))

                def rowf(r, _2):
                    bcast = lambda v: jnp.broadcast_to(v, (16,)).astype(jnp.int32)
                    coef = plsc.load_gather(p_v, [bcast(b), bcast(r)])
                    dloc = plsc.load_gather(idx_v, [bcast(r)])
                    coef = jnp.where(dloc < NHALF, coef, jnp.float32(0.0))
                    for j in range(n_groups):
                        sl = pl.ds(j * 16, 16)
                        rows_v[r, sl] = rows_v[r, sl] * coef
                    return _2
                lax.fori_loop(0, 128, rowf, 0)

                pltpu.sync_copy(rows_v, acc_sh.at[idx_v], add=True)
                return _
            lax.fori_loop(0, NB, batch, 0)

        plsc.subcore_barrier()

        rpt = NHALF // 16
        pltpu.sync_copy(
            acc_sh.at[pl.ds(sub * rpt, rpt)],
            out_h.at[pl.ds(core * NHALF + sub * rpt, rpt)])

    return agg


_agg128 = _make_agg(128)


def _decoder_body(a_ref, b_ref, o_ref):
    prod = jax.lax.dot_general(
        a_ref[...], b_ref[...], (((1,), (1,)), ((), ())),
        preferred_element_type=jnp.float32)
    o_ref[...] = jax.nn.sigmoid(prod)


def _decoder(h):
    N, K = h.shape
    hb = h.astype(jnp.bfloat16)
    BM = BN = 512
    grid = (pl.cdiv(N, BM), pl.cdiv(N, BN))
    return pl.pallas_call(
        _decoder_body,
        grid=grid,
        in_specs=[
            pl.BlockSpec((BM, K), lambda i, j: (i, 0)),
            pl.BlockSpec((BN, K), lambda i, j: (j, 0)),
        ],
        out_specs=pl.BlockSpec((BM, BN), lambda i, j: (i, j)),
        out_shape=jax.ShapeDtypeStruct((N, N), jnp.float32),
    )(hb, hb)


def _gat_sc(x, src3, dst3, W, att_src, att_dst, bias, Fpad):
    N = x.shape[0]
    F = W.shape[1]
    Wp = jnp.pad(W, ((0, 0), (0, Fpad - F)))
    att_s = jnp.pad(att_src, (0, Fpad - F))
    att_d = jnp.pad(att_dst, (0, Fpad - F))
    hp = x @ Wp
    a_src = hp @ att_s
    a_dst = hp @ att_d
    c = jax.nn.leaky_relu(jnp.max(a_src) + jnp.max(a_dst), negative_slope=0.2)
    c16 = jnp.broadcast_to(c, (16,))
    p3, den0, den1 = _edge_softmax(a_src, a_dst, c16, src3, dst3)
    denom = den0 + den1

    if Fpad == 128:
        agg = _agg128(hp, src3, dst3, p3)[:N]
    else:
        nch = Fpad // 128
        h8 = hp.reshape(N, nch, 128).transpose(1, 0, 2)
        chunks = []
        for ci in range(nch):
            chunks.append(_agg128(h8[ci], src3, dst3, p3)[:N])
        agg = jnp.concatenate(chunks, axis=1)

    out = agg / denom[:, None] + jnp.pad(bias, (0, Fpad - F))
    return jax.nn.relu(out)[:, :F]


def kernel(x, edge_index, W1, att_src1, att_dst1, b1, W2, att_src2, att_dst2, b2):
    N = x.shape[0]
    loop = jnp.arange(N, dtype=edge_index.dtype)
    src = jnp.concatenate([edge_index[0], loop])
    dst = jnp.concatenate([edge_index[1], loop])
    pad = jnp.zeros((E_PAD - E_REAL,), dtype=src.dtype)
    src3 = jnp.concatenate([src, pad]).reshape(32, NB, 128)
    dst3 = jnp.concatenate([dst, pad]).reshape(32, NB, 128)

    h = _gat_sc(x, src3, dst3, W1, att_src1, att_dst1, b1, 128)
    h = _gat_sc(h, src3, dst3, W2, att_src2, att_dst2, b2, 1536)
    adj = _decoder(h)
    return (adj, edge_index)


# speedup vs baseline: 3.0240x; 1.2541x over previous
import functools

import jax
import jax.numpy as jnp
from jax import lax
from jax.experimental import pallas as pl
from jax.experimental.pallas import tpu as pltpu
from jax.experimental.pallas import tpu_sc as plsc

N_NODES = 10000
E_REAL = 170000
NB = 42
E_PAD = 32 * NB * 128
_MESH = plsc.VectorSubcoreMesh(core_axis_name="c", subcore_axis_name="s")


@functools.partial(
    pl.kernel,
    mesh=_MESH,
    compiler_params=pltpu.CompilerParams(needs_layout_passes=False),
    out_type=(
        jax.ShapeDtypeStruct((32, NB, 128), jnp.float32),
        jax.ShapeDtypeStruct((N_NODES,), jnp.float32),
        jax.ShapeDtypeStruct((N_NODES,), jnp.float32),
    ),
    scratch_types=[
        pltpu.VMEM((N_NODES,), jnp.float32),
        pltpu.VMEM((N_NODES,), jnp.float32),
        pltpu.VMEM((16,), jnp.float32),
        pltpu.VMEM((NB, 128), jnp.int32),
        pltpu.VMEM((NB, 128), jnp.int32),
        pltpu.VMEM((NB, 128), jnp.float32),
        pltpu.VMEM((N_NODES,), jnp.float32),
        pltpu.VMEM_SHARED((N_NODES,), jnp.float32),
    ],
)
def _edge_softmax(a_src_h, a_dst_h, c_h, src_h, dst_h, p_out_h, den0_h, den1_h,
                  asrc_v, adst_v, c_v, src_v, dst_v, p_v, z_v, den_sh):
    core = lax.axis_index("c")
    sub = lax.axis_index("s")
    tid = core * 16 + sub

    pltpu.sync_copy(a_src_h, asrc_v)
    pltpu.sync_copy(a_dst_h, adst_v)
    pltpu.sync_copy(c_h, c_v)
    pltpu.sync_copy(src_h.at[tid], src_v)
    pltpu.sync_copy(dst_h.at[tid], dst_v)

    @pl.when(sub == 0)
    def _():
        def zb(i, _):
            z_v[pl.ds(i * 16, 16)] = jnp.zeros((16,), jnp.float32)
            return _
        lax.fori_loop(0, N_NODES // 16, zb, 0)
        pltpu.sync_copy(z_v, den_sh)

    cvec = c_v[...]
    base = tid * (NB * 128)

    def body(b, _):
        for j in range(8):
            sl = pl.ds(j * 16, 16)
            s_idx = src_v[b, sl]
            d_idx = dst_v[b, sl]
            av = plsc.load_gather(asrc_v, [s_idx])
            bv = plsc.load_gather(adst_v, [d_idx])
            al = av + bv
            al = jnp.where(al >= 0.0, al, al * jnp.float32(0.2))
            p = jnp.exp(al - cvec)
            eid = base + b * 128 + j * 16 + lax.iota(jnp.int32, 16)
            p = jnp.where(eid < E_REAL, p, jnp.float32(0.0))
            p_v[b, sl] = p
        return _

    lax.fori_loop(0, NB, body, 0)

    plsc.subcore_barrier()

    def addb(b, _):
        pltpu.sync_copy(p_v.at[b], den_sh.at[dst_v.at[b]], add=True)
        return _

    lax.fori_loop(0, NB, addb, 0)

    plsc.subcore_barrier()

    pltpu.sync_copy(p_v, p_out_h.at[tid])

    @pl.when(jnp.logical_and(sub == 0, core == 0))
    def _():
        pltpu.sync_copy(den_sh, den0_h)

    @pl.when(jnp.logical_and(sub == 0, core == 1))
    def _():
        pltpu.sync_copy(den_sh, den1_h)


NHALF = 5120
NROWS_ACC = 5248


def _make_agg(F):
    n_groups = F // 16

    @functools.partial(
        pl.kernel,
        mesh=_MESH,
        compiler_params=pltpu.CompilerParams(needs_layout_passes=False),
        out_type=jax.ShapeDtypeStruct((2 * NHALF, F), jnp.float32),
        scratch_types=[
            pltpu.VMEM((NB, 128), jnp.int32),
            pltpu.VMEM((NB, 128), jnp.int32),
            pltpu.VMEM((NB, 128), jnp.float32),
            pltpu.VMEM((NB, 128), jnp.int32),
            pltpu.VMEM((128, F), jnp.float32),
            pltpu.VMEM((128, F), jnp.float32),
            pltpu.VMEM((128, F), jnp.float32),
            pltpu.VMEM_SHARED((NROWS_ACC, F), jnp.float32),
            pltpu.SemaphoreType.DMA,
            pltpu.SemaphoreType.DMA,
        ],
    )
    def agg(h_h, src_h, dst_h, p_h, out_h, src_v, dst_v, p_v, idx_v,
            rows0_v, rows1_v, z_v, acc_sh, sem0, sem1):
        core = lax.axis_index("c")
        sub = lax.axis_index("s")
        lo = core * NHALF
        zr = NROWS_ACC // 16

        def zrow(r, _):
            for j in range(n_groups):
                z_v[r, pl.ds(j * 16, 16)] = jnp.zeros((16,), jnp.float32)
            return _
        lax.fori_loop(0, 128, zrow, 0)
        base = sub * zr
        pltpu.sync_copy(z_v, acc_sh.at[pl.ds(base, 128)])
        pltpu.sync_copy(z_v, acc_sh.at[pl.ds(base + 128, 128)])
        pltpu.sync_copy(z_v.at[pl.ds(0, zr - 256)],
                        acc_sh.at[pl.ds(base + 256, zr - 256)])

        plsc.subcore_barrier()

        rows = (rows0_v, rows1_v)
        sems = (sem0, sem1)

        def process(b, rv):
            def rowf(r2, _2):
                for u in range(2):
                    r = r2 * 2 + u
                    bcast = lambda v: jnp.broadcast_to(v, (16,)).astype(jnp.int32)
                    coef = plsc.load_gather(p_v, [bcast(b), bcast(r)])
                    for j in range(n_groups):
                        sl = pl.ds(j * 16, 16)
                        rv[r, sl] = rv[r, sl] * coef
                return _2
            lax.fori_loop(0, 64, rowf, 0)
            pltpu.sync_copy(rv, acc_sh.at[idx_v.at[b]], add=True)

        for ec in range(2):
            cid = sub * 2 + ec
            pltpu.sync_copy(src_h.at[cid], src_v)
            pltpu.sync_copy(dst_h.at[cid], dst_v)
            pltpu.sync_copy(p_h.at[cid], p_v)

            def pre(b, _):
                for g in range(8):
                    sl = pl.ds(g * 16, 16)
                    loc = dst_v[b, sl] - lo
                    inb = jnp.logical_and(loc >= 0, loc < NHALF)
                    idx_v[b, sl] = jnp.where(inb, loc, NHALF + g * 16
                                             + lax.iota(jnp.int32, 16))
                    p_v[b, sl] = jnp.where(inb, p_v[b, sl], jnp.float32(0.0))
                return _
            lax.fori_loop(0, NB, pre, 0)

            pltpu.async_copy(h_h.at[src_v.at[0]], rows0_v, sem0)

            def pair(bb, _):
                b0 = bb * 2
                b1 = b0 + 1
                pltpu.make_async_copy(h_h.at[src_v.at[b0]], rows0_v, sem0).wait()
                pltpu.async_copy(h_h.at[src_v.at[b1]], rows1_v, sem1)
                process(b0, rows0_v)
                nxt = jnp.minimum(b0 + 2, NB - 1)
                pltpu.make_async_copy(h_h.at[src_v.at[b1]], rows1_v, sem1).wait()
                pltpu.async_copy(h_h.at[src_v.at[nxt]], rows0_v, sem0)
                process(b1, rows1_v)
                return _
            lax.fori_loop(0, NB // 2, pair, 0)
            pltpu.make_async_copy(h_h.at[src_v.at[0]], rows0_v, sem0).wait()

        plsc.subcore_barrier()

        rpt = NHALF // 16
        pltpu.sync_copy(
            acc_sh.at[pl.ds(sub * rpt, rpt)],
            out_h.at[pl.ds(core * NHALF + sub * rpt, rpt)])

    return agg


_agg128 = _make_agg(128)


def _decoder_body(a_ref, b_ref, o_ref):
    prod = jax.lax.dot_general(
        a_ref[...], b_ref[...], (((1,), (1,)), ((), ())),
        preferred_element_type=jnp.float32)
    o_ref[...] = jax.nn.sigmoid(prod)


def _decoder(h):
    N, K = h.shape
    hb = h.astype(jnp.bfloat16)
    BM = BN = 512
    grid = (pl.cdiv(N, BM), pl.cdiv(N, BN))
    return pl.pallas_call(
        _decoder_body,
        grid=grid,
        in_specs=[
            pl.BlockSpec((BM, K), lambda i, j: (i, 0)),
            pl.BlockSpec((BN, K), lambda i, j: (j, 0)),
        ],
        out_specs=pl.BlockSpec((BM, BN), lambda i, j: (i, j)),
        out_shape=jax.ShapeDtypeStruct((N, N), jnp.float32),
    )(hb, hb)


def _gat_sc(x, src3, dst3, W, att_src, att_dst, bias, Fpad):
    N = x.shape[0]
    F = W.shape[1]
    Wp = jnp.pad(W, ((0, 0), (0, Fpad - F)))
    att_s = jnp.pad(att_src, (0, Fpad - F))
    att_d = jnp.pad(att_dst, (0, Fpad - F))
    hp = x @ Wp
    a_src = hp @ att_s
    a_dst = hp @ att_d
    c = jax.nn.leaky_relu(jnp.max(a_src) + jnp.max(a_dst), negative_slope=0.2)
    c16 = jnp.broadcast_to(c, (16,))
    p3, den0, den1 = _edge_softmax(a_src, a_dst, c16, src3, dst3)
    denom = den0 + den1

    if Fpad == 128:
        agg = _agg128(hp, src3, dst3, p3)[:N]
    else:
        nch = Fpad // 128
        chunks = []
        for ci in range(nch):
            tab = hp[:, ci * 128:(ci + 1) * 128]
            chunks.append(_agg128(tab, src3, dst3, p3)[:N])
        agg = jnp.concatenate(chunks, axis=1)

    out = agg / denom[:, None] + jnp.pad(bias, (0, Fpad - F))
    return jax.nn.relu(out)[:, :F]


def kernel(x, edge_index, W1, att_src1, att_dst1, b1, W2, att_src2, att_dst2, b2):
    N = x.shape[0]
    loop = jnp.arange(N, dtype=edge_index.dtype)
    src = jnp.concatenate([edge_index[0], loop])
    dst = jnp.concatenate([edge_index[1], loop])
    pad = jnp.zeros((E_PAD - E_REAL,), dtype=src.dtype)
    src3 = jnp.concatenate([src, pad]).reshape(32, NB, 128)
    dst3 = jnp.concatenate([dst, pad]).reshape(32, NB, 128)

    h = _gat_sc(x, src3, dst3, W1, att_src1, att_dst1, b1, 128)
    h = _gat_sc(h, src3, dst3, W2, att_src2, att_dst2, b2, 1536)
    adj = _decoder(h)
    return (adj, edge_index)
